# baseline (speedup 1.0000x reference)
import functools

import jax
import jax.numpy as jnp
from jax import lax
from jax.experimental import pallas as pl
from jax.experimental.pallas import tpu as pltpu
from jax.experimental.pallas import tpu_sc as plsc

N_NODES = 10000
N_EDGES = 160000
NUM_RELS = 19
DIM = 256

BN = 1000
NB = N_NODES // BN


def _xw_body(h_ref, w_ref, o_ref):
    o_ref[...] = jnp.dot(h_ref[...].astype(jnp.bfloat16),
                         w_ref[0].astype(jnp.bfloat16),
                         preferred_element_type=jnp.float32)


def _xw_table(h, W):
    return pl.pallas_call(
        _xw_body,
        grid=(NB, NUM_RELS),
        in_specs=[
            pl.BlockSpec((BN, DIM), lambda nb, r: (nb, 0)),
            pl.BlockSpec((1, DIM, DIM), lambda nb, r: (r, 0, 0)),
        ],
        out_specs=pl.BlockSpec((BN, DIM), lambda nb, r: (r * NB + nb, 0)),
        out_shape=jax.ShapeDtypeStruct((NUM_RELS * N_NODES, DIM), jnp.float32),
    )(h, W)


def _act_body(a_ref, b_ref, o_ref, *, relu):
    v = a_ref[...] + b_ref[...]
    if relu:
        v = jnp.maximum(v, 0.0)
    o_ref[...] = v


def _bias_act(a, b, relu):
    return pl.pallas_call(
        functools.partial(_act_body, relu=relu),
        grid=(NB,),
        in_specs=[
            pl.BlockSpec((BN, DIM), lambda nb: (nb, 0)),
            pl.BlockSpec((1, DIM), lambda nb: (0, 0)),
        ],
        out_specs=pl.BlockSpec((BN, DIM), lambda nb: (nb, 0)),
        out_shape=jax.ShapeDtypeStruct((N_NODES, DIM), jnp.float32),
    )(a, b.reshape(1, DIM))


LANES = 16
N_TILES = 32
TPT = 320
TRASH = TPT
ACC_ROWS = 328
CHUNK = 64
PADT = N_EDGES + N_TILES * CHUNK

def _prep_edges(src, et, dst):
    bucket = (dst // TPT).astype(jnp.int32)
    oh = (jnp.arange(N_TILES, dtype=jnp.int32)[:, None] == bucket[None, :])
    pref = jnp.cumsum(oh.astype(jnp.int32), axis=1)
    rank = jnp.take_along_axis(pref, bucket[None, :], 0)[0] - 1
    counts = pref[:, -1]
    pcount = ((counts + CHUNK - 1) // CHUNK) * CHUNK
    pstarts = (jnp.cumsum(pcount) - pcount).astype(jnp.int32)
    pos = pstarts[bucket] + rank
    rid = et * N_NODES + src
    ppos = pos // CHUNK * (2 * CHUNK) + pos % CHUNK
    slot = jnp.arange(2 * PADT, dtype=jnp.int32) // CHUNK % 2
    init = jnp.where(slot == 1, TRASH, 0).astype(jnp.int32)
    both_pos = jnp.concatenate([ppos, ppos + CHUNK])
    both_val = jnp.concatenate([rid, dst - bucket * TPT])
    pidx = init.at[both_pos].set(both_val)
    return pidx, pstarts, (pcount // CHUNK).astype(jnp.int32)


@functools.cache
def _sc_agg_kernel():
    mesh = plsc.VectorSubcoreMesh(core_axis_name="c", subcore_axis_name="s")
    return pl.kernel(
        _sc_agg_body,
        mesh=mesh,
        compiler_params=pltpu.CompilerParams(needs_layout_passes=False),
        out_type=jax.ShapeDtypeStruct((N_NODES, DIM), jnp.float32),
        scratch_types=[
            pltpu.VMEM((N_TILES,), jnp.int32),
            pltpu.VMEM((N_TILES,), jnp.int32),
            pltpu.VMEM((2 * CHUNK,), jnp.int32),
            pltpu.VMEM((2 * CHUNK,), jnp.int32),
            pltpu.VMEM((CHUNK, DIM), jnp.float32),
            pltpu.VMEM((CHUNK, DIM), jnp.float32),
            pltpu.VMEM((ACC_ROWS, DIM), jnp.float32),
            pltpu.SemaphoreType.DMA,
            pltpu.SemaphoreType.DMA,
        ],
    )


def _sc_agg(table, pidx, pstarts, nchunks):
    return _sc_agg_kernel()(table, pidx, pstarts, nchunks)


def _sc_agg_body(table, pidx_hbm, pst_hbm, nch_hbm, out_hbm,
                 pst_v, nch_v, idx0, idx1, rows0, rows1, acc, gsem0, gsem1):
    c = lax.axis_index("c")
    s = lax.axis_index("s")
    lane = jax.lax.iota(jnp.int32, 16)

    pltpu.sync_copy(pst_hbm, pst_v)
    pltpu.sync_copy(nch_hbm, nch_v)
    meta_sl = pl.ds(pl.multiple_of(c * 16, 16), 16)
    pstart = jnp.max(jnp.where(lane == s, pst_v[meta_sl], -1))
    nch = jnp.max(jnp.where(lane == s, nch_v[meta_sl], -1))

    def _zero_row(r, carry):
        for k in range(DIM // LANES):
            acc[r, pl.ds(k * LANES, LANES)] = jnp.zeros((LANES,), jnp.float32)
        return carry
    lax.fori_loop(0, ACC_ROWS, _zero_row, 0)

    idxs = (idx0, idx1)
    rowss = (rows0, rows1)
    gsems = (gsem0, gsem1)

    def _stage(i, b):
        base2 = pl.multiple_of((pstart + i * CHUNK) * 2, 2 * CHUNK)
        pltpu.sync_copy(pidx_hbm.at[pl.ds(base2, 2 * CHUNK)], idxs[b])
        pltpu.async_copy(table.at[idxs[b].at[pl.ds(0, CHUNK)]],
                         rowss[b], gsems[b])

    def _wait(b):
        pltpu.make_async_copy(table.at[idxs[b].at[pl.ds(0, CHUNK)]],
                              rowss[b], gsems[b]).wait()

    def _accum(b):
        def _group(g, carry2):
            goff = pl.multiple_of(g * LANES, LANES)
            dg = idxs[b][pl.ds(CHUNK + goff, LANES)]
            for j2 in range(LANES):
                dl_splat = dg[jnp.full((LANES,), j2, jnp.int32)]
                j = goff + j2
                for k in range(DIM // LANES):
                    sl = pl.ds(k * LANES, LANES)
                    colv = lane + (k * LANES)
                    plsc.addupdate_scatter(acc, [dl_splat, colv],
                                           rowss[b][j, sl])
            return carry2
        lax.fori_loop(0, CHUNK // LANES, _group, 0)

    @pl.when(nch > 0)
    def _prime():
        _stage(0, 0)

    def _pair(t, carry):
        i1 = 2 * t + 1

        @pl.when(i1 < nch)
        def _s1():
            _stage(i1, 1)
        _wait(0)
        _accum(0)

        @pl.when(i1 + 1 < nch)
        def _s0():
            _stage(i1 + 1, 0)

        @pl.when(i1 < nch)
        def _w1():
            _wait(1)
            _accum(1)
        return carry
    lax.fori_loop(0, (nch + 1) // 2, _pair, 0)

    w = c * 16 + s
    obase = w * TPT

    @pl.when(w < N_TILES - 1)
    def _copy_full():
        pltpu.sync_copy(acc.at[pl.ds(0, TPT)], out_hbm.at[pl.ds(obase, TPT)])

    @pl.when(w == N_TILES - 1)
    def _copy_tail():
        tail = N_NODES - (N_TILES - 1) * TPT
        pltpu.sync_copy(acc.at[pl.ds(0, tail)], out_hbm.at[pl.ds(obase, tail)])


def kernel(x, edge_index, edge_type, W0, b0, W1, b1, W2, b2):
    src = edge_index[0]
    dst = edge_index[1]
    pidx, pstarts, nchunks = _prep_edges(src, edge_type, dst)
    h = x
    for l, (W, b) in enumerate(((W0, b0), (W1, b1), (W2, b2))):
        table = _xw_table(h, W)
        agg = _sc_agg(table, pidx, pstarts, nchunks)
        h = _bias_act(agg, b, relu=(l < 2))
    return h
